# Initial kernel scaffold; baseline (speedup 1.0000x reference)
import jax
import jax.numpy as jnp
from jax.experimental import pallas as pl


def kernel(emb_sentences, att_sentences, W):
    raise NotImplementedError("write your pallas kernel here")


# speedup vs baseline: 2.8474x; 2.8474x over previous
import jax
import jax.numpy as jnp
from jax.experimental import pallas as pl


def _body(emb_ref, att_ref, w_ref, out_ref):
    L, D = emb_ref.shape[1], emb_ref.shape[2]
    NL = w_ref.shape[0]
    e = emb_ref[0]
    w1 = w_ref[:, :D]
    w2 = w_ref[:, D:]
    a = jax.lax.dot_general(e, w1, (((1,), (1,)), ((), ())),
                            preferred_element_type=jnp.float32)
    bv = jax.lax.dot_general(e, w2, (((1,), (1,)), ((), ())),
                             preferred_element_type=jnp.float32)
    attc = att_ref[0]
    neg_inf = jnp.float32(-jnp.inf)
    a = jnp.where(attc > 0, a, neg_inf)
    bv = jnp.where(attc > 0, bv, neg_inf)
    JC = 8
    for jc in range(L // JC):
        bchunk = bv[jc * JC:(jc + 1) * JC]
        blk = a[None, :, :] + bchunk[:, None, :]
        jg = jc * JC + jax.lax.broadcasted_iota(jnp.int32, (JC, L, 1), 0)
        kg = jax.lax.broadcasted_iota(jnp.int32, (JC, L, 1), 1)
        blk = jnp.where(jg == kg, neg_inf, blk)
        out_ref[0, jc * JC:(jc + 1) * JC] = blk


def kernel(emb_sentences, att_sentences, W):
    B, L, D = emb_sentences.shape
    NL = W.shape[0]
    att_col = att_sentences.astype(jnp.float32).reshape(B, L, 1)
    out4 = pl.pallas_call(
        _body,
        grid=(B,),
        in_specs=[
            pl.BlockSpec((1, L, D), lambda b: (b, 0, 0)),
            pl.BlockSpec((1, L, 1), lambda b: (b, 0, 0)),
            pl.BlockSpec((NL, 2 * D), lambda b: (0, 0)),
        ],
        out_specs=pl.BlockSpec((1, L, L, NL), lambda b: (b, 0, 0, 0)),
        out_shape=jax.ShapeDtypeStruct((B, L, L, NL), jnp.float32),
    )(emb_sentences, att_col, W)
    return out4.reshape(B, L * L, NL)
